# baseline (speedup 1.0000x reference)
import jax
import jax.numpy as jnp
from jax import lax
from jax.experimental import pallas as pl
from jax.experimental.pallas import tpu as pltpu
from jax.experimental.pallas import tpu_sc as plsc

_NUM_CHARS = 128
_L = 20
_E = 64
_WORDS = 256 * 50
_NW = 32
_WPT = _WORDS // _NW
_GROUPS = _WPT // 16
_OUT_D = 3 * _E


def _sc_body(sntcs_hbm, w_hbm, out_hbm, chars_v, w_v, out_v):
    wid = lax.axis_index("s") * 2 + lax.axis_index("c")
    pltpu.sync_copy(sntcs_hbm.at[pl.ds(wid * (_WPT * _L), _WPT * _L)], chars_v)
    pltpu.sync_copy(w_hbm, w_v)

    lane = lax.iota(jnp.int32, 16)
    lane_l = lane * _L
    lane_d = lane * _OUT_D

    def group(g, carry):
        cbase = lane_l + g * (16 * _L)
        c = [plsc.load_gather(chars_v, [cbase + l]) for l in range(_L)]

        m = c[0]
        a = jnp.zeros((16,), jnp.int32)
        for l in range(1, _L):
            gt = c[l] > m
            a = jnp.where(gt, l, a)
            m = jnp.where(gt, c[l], m)
        wl = jnp.maximum(a - 1, 0)
        ends = plsc.load_gather(chars_v, [cbase + wl])

        rows = [jnp.where(wl == 0, 0, c[0])]
        rows += [jnp.where(wl == l, 0, c[l]) for l in range(1, _L - 1)]
        rows.append(ends)

        wbase = lane_d + g * (16 * _OUT_D)

        def ebody(e, carry2):
            e128 = e * _NUM_CHARS
            first = plsc.load_gather(w_v, [rows[0] + e128])
            acc = plsc.load_gather(w_v, [rows[1] + e128])
            for l in range(2, _L - 1):
                acc = acc + plsc.load_gather(w_v, [rows[l] + e128])
            lastv = plsc.load_gather(w_v, [rows[_L - 1] + e128])
            plsc.store_scatter(out_v, [wbase + e], first)
            plsc.store_scatter(out_v, [wbase + (e + _E)], acc)
            plsc.store_scatter(out_v, [wbase + (e + 2 * _E)], lastv)
            return carry2

        lax.fori_loop(0, _E, ebody, 0)
        return carry

    lax.fori_loop(0, _GROUPS, group, 0)
    pltpu.sync_copy(out_v, out_hbm.at[pl.ds(wid * (_WPT * _OUT_D), _WPT * _OUT_D)])


def kernel(sntcs, W):
    s_flat = sntcs.reshape(-1).astype(jnp.int32)
    w_flat = W.reshape(-1)
    mesh = plsc.VectorSubcoreMesh(core_axis_name="c", subcore_axis_name="s")
    run = pl.kernel(
        _sc_body,
        mesh=mesh,
        compiler_params=pltpu.CompilerParams(needs_layout_passes=False),
        out_type=jax.ShapeDtypeStruct((_WORDS * _OUT_D,), jnp.float32),
        scratch_types=[
            pltpu.VMEM((_WPT * _L,), jnp.int32),
            pltpu.VMEM((_NUM_CHARS * _E,), jnp.float32),
            pltpu.VMEM((_WPT * _OUT_D,), jnp.float32),
        ],
    )
    out = run(s_flat, w_flat)
    return out.reshape(256, 50, _OUT_D)
